# baseline (device time: 123937 ns/iter reference)
import jax
import jax.numpy as jnp
from jax import lax
from jax.experimental import pallas as pl
from jax.experimental.pallas import tpu as pltpu

N_DEV = 4
SQ = 1024
SKV = 1024
KVH = SKV // 4
HQ = 8
DH = 128
DM = HQ * DH
SCALE = 0.08838834764831843
BLK = 64

FROM_L, FROM_R, FROM_D = range(3)


def _body(x_ref, wq_ref, k_ref, v_ref, wo_ref, out_ref,
          comm_ref, q_ref, acc_ref, l_ref, bias_ref, send_sems, recv_sems):
    my = lax.axis_index("i")
    left = lax.rem(my + N_DEV - 1, N_DEV)
    right = lax.rem(my + 1, N_DEV)
    diag = lax.rem(my + 2, N_DEV)

    barrier_sem = pltpu.get_barrier_semaphore()
    for nbr in (left, right):
        pl.semaphore_signal(barrier_sem, inc=1, device_id=(nbr,),
                            device_id_type=pl.DeviceIdType.MESH)
    pl.semaphore_wait(barrier_sem, 2)

    send_rk = pltpu.make_async_remote_copy(
        src_ref=k_ref, dst_ref=comm_ref.at[FROM_L, pl.ds(0, HQ)],
        send_sem=send_sems.at[0], recv_sem=recv_sems.at[0],
        device_id=(right,), device_id_type=pl.DeviceIdType.MESH)
    send_rv = pltpu.make_async_remote_copy(
        src_ref=v_ref, dst_ref=comm_ref.at[FROM_L, pl.ds(HQ, HQ)],
        send_sem=send_sems.at[1], recv_sem=recv_sems.at[1],
        device_id=(right,), device_id_type=pl.DeviceIdType.MESH)
    send_lk = pltpu.make_async_remote_copy(
        src_ref=k_ref, dst_ref=comm_ref.at[FROM_R, pl.ds(0, HQ)],
        send_sem=send_sems.at[2], recv_sem=recv_sems.at[2],
        device_id=(left,), device_id_type=pl.DeviceIdType.MESH)
    send_lv = pltpu.make_async_remote_copy(
        src_ref=v_ref, dst_ref=comm_ref.at[FROM_R, pl.ds(HQ, HQ)],
        send_sem=send_sems.at[3], recv_sem=recv_sems.at[3],
        device_id=(left,), device_id_type=pl.DeviceIdType.MESH)
    send_rk.start()
    send_lk.start()
    send_rv.start()
    send_lv.start()

    def qproj(hd, carry):
        c = hd * DH
        qh = lax.dot(x_ref[...], wq_ref[:, pl.ds(c, DH)],
                     preferred_element_type=jnp.float32)
        q_ref[:, pl.ds(c, DH)] = qh.astype(jnp.bfloat16)
        return carry
    lax.fori_loop(0, HQ, qproj, 0)

    qbv = (my * SQ + lax.broadcasted_iota(jnp.int32, (SQ, 1), 0)) // BLK
    ones_kd = jnp.ones((KVH, DH), jnp.bfloat16)

    def accumulate(khead, vhead, src, init=False):
        for bh in range(4):
            bo = bh * KVH
            kbv = (src * SKV + bo
                   + lax.broadcasted_iota(jnp.int32, (1, KVH), 1)) // BLK
            keep = (qbv == kbv) | (kbv == 0) | (lax.rem(qbv + kbv, 3) == 0)
            bias_ref[:, bo:bo + KVH] = jnp.where(keep, 0.0, -40.0)

        def head_step(hd, carry):
            c = hd * DH
            qs = q_ref[:, pl.ds(c, DH)]
            for half in range(4):
                o = half * KVH
                s = lax.dot_general(qs, khead(hd, o),
                                    (((1,), (1,)), ((), ())),
                                    preferred_element_type=jnp.float32)
                w = jnp.exp((s + bias_ref[:, o:o + KVH]).astype(
                    jnp.bfloat16)).astype(jnp.bfloat16)
                lsum = lax.dot(w, ones_kd, preferred_element_type=jnp.float32)
                ctx = lax.dot(w, vhead(hd, o),
                              preferred_element_type=jnp.float32)
                if init and half == 0:
                    l_ref[hd] = lsum
                    acc_ref[:, pl.ds(c, DH)] = ctx
                else:
                    l_ref[hd] = l_ref[hd] + lsum
                    acc_ref[:, pl.ds(c, DH)] = acc_ref[:, pl.ds(c, DH)] + ctx
            return carry
        lax.fori_loop(0, HQ, head_step, 0)

    def slot_kv(slot):
        return (lambda hd, o: comm_ref[slot, hd, pl.ds(o, KVH)],
                lambda hd, o: comm_ref[slot, HQ + hd, pl.ds(o, KVH)])

    accumulate(lambda hd, o: k_ref[hd, pl.ds(o, KVH)],
               lambda hd, o: v_ref[hd, pl.ds(o, KVH)], my, init=True)

    send_rk.wait()
    send_rv.wait()
    fwd_k = pltpu.make_async_remote_copy(
        src_ref=comm_ref.at[FROM_L, pl.ds(0, HQ)],
        dst_ref=comm_ref.at[FROM_D, pl.ds(0, HQ)],
        send_sem=send_sems.at[4], recv_sem=recv_sems.at[4],
        device_id=(right,), device_id_type=pl.DeviceIdType.MESH)
    fwd_k.start()
    accumulate(*slot_kv(FROM_L), left)

    send_lk.wait()
    send_lv.wait()
    fwd_v = pltpu.make_async_remote_copy(
        src_ref=comm_ref.at[FROM_R, pl.ds(HQ, HQ)],
        dst_ref=comm_ref.at[FROM_D, pl.ds(HQ, HQ)],
        send_sem=send_sems.at[5], recv_sem=recv_sems.at[5],
        device_id=(left,), device_id_type=pl.DeviceIdType.MESH)
    fwd_v.start()
    accumulate(*slot_kv(FROM_R), right)

    fwd_k.wait()
    fwd_v.wait()
    accumulate(*slot_kv(FROM_D), diag)

    def norm(hd, carry):
        c = hd * DH
        q_ref[:, pl.ds(c, DH)] = (
            acc_ref[:, pl.ds(c, DH)] / l_ref[hd]).astype(jnp.bfloat16)
        return carry
    lax.fori_loop(0, HQ, norm, 0)

    out_ref[...] = lax.dot(q_ref[...], wo_ref[...],
                           preferred_element_type=jnp.float32)


def kernel(x, Wq, K_ext, V_ext, Wo):
    xb = x[0].astype(jnp.bfloat16)
    wq = (Wq * SCALE).astype(jnp.bfloat16)
    wo = Wo.astype(jnp.bfloat16)
    k2 = jnp.transpose(K_ext[0], (1, 0, 2)).astype(jnp.bfloat16)
    v2 = jnp.transpose(V_ext[0], (1, 0, 2)).astype(jnp.bfloat16)

    out = pl.pallas_call(
        _body,
        out_shape=jax.ShapeDtypeStruct((SQ, DM), jnp.float32),
        in_specs=[pl.BlockSpec(memory_space=pltpu.VMEM)] * 5,
        out_specs=pl.BlockSpec(memory_space=pltpu.VMEM),
        scratch_shapes=[
            pltpu.VMEM((3, 2 * HQ, SKV, DH), jnp.bfloat16),
            pltpu.VMEM((SQ, DM), jnp.bfloat16),
            pltpu.VMEM((SQ, DM), jnp.float32),
            pltpu.VMEM((HQ, SQ, DH), jnp.float32),
            pltpu.VMEM((SQ, SKV), jnp.float32),
            pltpu.SemaphoreType.DMA((6,)),
            pltpu.SemaphoreType.DMA((6,)),
        ],
        compiler_params=pltpu.CompilerParams(collective_id=0),
    )(xb, wq, k2, v2, wo)
    return out[None]
